# baseline (speedup 1.0000x reference)
import jax
import jax.numpy as jnp
from jax import lax
from jax.experimental import pallas as pl
from jax.experimental.pallas import tpu as pltpu

N = 10000
D = 128
E = 320000
EB = 1000
NEB = E // EB

f32 = jnp.float32


def _make_agg(with_cnt: bool):
  def body(src_r, dst_r, x_r, agg_r, *maybe_cnt):
    @pl.when(pl.program_id(0) == 0)
    def _init():
      agg_r[...] = jnp.zeros((N, D), f32)
      if with_cnt:
        maybe_cnt[0][...] = jnp.zeros((N, D), f32)

    one = jnp.ones((1, D), f32)

    def edge(e, carry):
      s = src_r[0, 0, e]
      d = dst_r[0, 0, e]
      agg_r[pl.ds(d, 1), :] += x_r[pl.ds(s, 1), :]
      if with_cnt:
        maybe_cnt[0][pl.ds(d, 1), :] += one
      return carry

    lax.fori_loop(0, EB, edge, 0)

  out_shape = [jax.ShapeDtypeStruct((N, D), f32)]
  out_specs = [pl.BlockSpec((N, D), lambda i: (0, 0))]
  if with_cnt:
    out_shape.append(jax.ShapeDtypeStruct((N, D), f32))
    out_specs.append(pl.BlockSpec((N, D), lambda i: (0, 0)))

  return pl.pallas_call(
      body,
      grid=(NEB,),
      in_specs=[
          pl.BlockSpec((1, 1, EB), lambda i: (i, 0, 0),
                       memory_space=pltpu.SMEM),
          pl.BlockSpec((1, 1, EB), lambda i: (i, 0, 0),
                       memory_space=pltpu.SMEM),
          pl.BlockSpec((N, D), lambda i: (0, 0)),
      ],
      out_specs=out_specs,
      out_shape=out_shape,
  )


_agg_with_cnt = _make_agg(True)
_agg_no_cnt = _make_agg(False)


def _make_tc_layer(with_gelu: bool):
  BN = 1000

  def body(a_r, c_r, x_r, wl_r, b_r, wr_r, o_r):
    mean = a_r[...] / jnp.maximum(c_r[:, 0:1], 1.0)
    h = (jnp.dot(mean, wl_r[...], preferred_element_type=f32) + b_r[...]
         + jnp.dot(x_r[...], wr_r[...], preferred_element_type=f32))
    if with_gelu:
      h = h * 0.5 * (1.0 + lax.erf(h * (2.0 ** -0.5)))
    o_r[...] = h

  return pl.pallas_call(
      body,
      grid=(N // BN,),
      in_specs=[
          pl.BlockSpec((BN, D), lambda i: (i, 0)),
          pl.BlockSpec((BN, D), lambda i: (i, 0)),
          pl.BlockSpec((BN, D), lambda i: (i, 0)),
          pl.BlockSpec((D, D), lambda i: (0, 0)),
          pl.BlockSpec((1, D), lambda i: (0, 0)),
          pl.BlockSpec((D, D), lambda i: (0, 0)),
      ],
      out_specs=pl.BlockSpec((BN, D), lambda i: (i, 0)),
      out_shape=jax.ShapeDtypeStruct((N, D), f32),
  )


_tc_gelu = _make_tc_layer(True)
_tc_plain = _make_tc_layer(False)


def kernel(embs, edge_index, W1l, b1l, W1r, W2l, b2l, W2r):
  src3 = edge_index[0].reshape(NEB, 1, EB)
  dst3 = edge_index[1].reshape(NEB, 1, EB)

  agg1, cnt = _agg_with_cnt(src3, dst3, embs)
  z = _tc_gelu(agg1, cnt, embs, W1l.T, b1l.reshape(1, D), W1r.T)
  (agg2,) = _agg_no_cnt(src3, dst3, z)
  out = _tc_plain(agg2, cnt, z, W2l.T, b2l.reshape(1, D), W2r.T)
  return out


# speedup vs baseline: 1.6920x; 1.6920x over previous
import jax
import jax.numpy as jnp
from jax import lax
from jax.experimental import pallas as pl
from jax.experimental.pallas import tpu as pltpu

N = 10000
D = 128
E = 320000
EB = 1000
NEB = E // EB

f32 = jnp.float32


def _make_agg(with_cnt: bool):
  def body(src_r, dst_r, x_r, agg_r, *maybe_cnt):
    @pl.when(pl.program_id(0) == 0)
    def _init():
      agg_r[...] = jnp.zeros((N, D), f32)
      if with_cnt:
        maybe_cnt[0][...] = jnp.zeros((N, D), f32)

    one = jnp.ones((1, D), f32)

    def edge(e, carry):
      for u in range(4):
        e4 = e * 4 + u
        s = src_r[0, 0, e4]
        d = dst_r[0, 0, e4]
        agg_r[pl.ds(d, 1), :] += x_r[pl.ds(s, 1), :]
        if with_cnt:
          maybe_cnt[0][pl.ds(d, 1), :] += one
      return carry

    lax.fori_loop(0, EB // 4, edge, 0)

  out_shape = [jax.ShapeDtypeStruct((N, D), f32)]
  out_specs = [pl.BlockSpec((N, D), lambda i: (0, 0))]
  if with_cnt:
    out_shape.append(jax.ShapeDtypeStruct((N, D), f32))
    out_specs.append(pl.BlockSpec((N, D), lambda i: (0, 0)))

  return pl.pallas_call(
      body,
      grid=(NEB,),
      in_specs=[
          pl.BlockSpec((1, 1, EB), lambda i: (i, 0, 0),
                       memory_space=pltpu.SMEM),
          pl.BlockSpec((1, 1, EB), lambda i: (i, 0, 0),
                       memory_space=pltpu.SMEM),
          pl.BlockSpec((N, D), lambda i: (0, 0)),
      ],
      out_specs=out_specs,
      out_shape=out_shape,
  )


_agg_with_cnt = _make_agg(True)
_agg_no_cnt = _make_agg(False)


def _make_tc_layer(with_gelu: bool):
  BN = 1000

  def body(a_r, c_r, x_r, wl_r, b_r, wr_r, o_r):
    mean = a_r[...] / jnp.maximum(c_r[:, 0:1], 1.0)
    h = (jnp.dot(mean, wl_r[...], preferred_element_type=f32) + b_r[...]
         + jnp.dot(x_r[...], wr_r[...], preferred_element_type=f32))
    if with_gelu:
      h = h * 0.5 * (1.0 + lax.erf(h * (2.0 ** -0.5)))
    o_r[...] = h

  return pl.pallas_call(
      body,
      grid=(N // BN,),
      in_specs=[
          pl.BlockSpec((BN, D), lambda i: (i, 0)),
          pl.BlockSpec((BN, D), lambda i: (i, 0)),
          pl.BlockSpec((BN, D), lambda i: (i, 0)),
          pl.BlockSpec((D, D), lambda i: (0, 0)),
          pl.BlockSpec((1, D), lambda i: (0, 0)),
          pl.BlockSpec((D, D), lambda i: (0, 0)),
      ],
      out_specs=pl.BlockSpec((BN, D), lambda i: (i, 0)),
      out_shape=jax.ShapeDtypeStruct((N, D), f32),
  )


_tc_gelu = _make_tc_layer(True)
_tc_plain = _make_tc_layer(False)


def kernel(embs, edge_index, W1l, b1l, W1r, W2l, b2l, W2r):
  src3 = edge_index[0].reshape(NEB, 1, EB)
  dst3 = edge_index[1].reshape(NEB, 1, EB)

  agg1, cnt = _agg_with_cnt(src3, dst3, embs)
  z = _tc_gelu(agg1, cnt, embs, W1l.T, b1l.reshape(1, D), W1r.T)
  (agg2,) = _agg_no_cnt(src3, dst3, z)
  out = _tc_plain(agg2, cnt, z, W2l.T, b2l.reshape(1, D), W2r.T)
  return out
